# baseline (speedup 1.0000x reference)
import jax
import jax.numpy as jnp
from jax.experimental import pallas as pl
from jax.experimental.pallas import tpu as pltpu

B, H, N, D = 8, 12, 577, 768
K = 288
NP = N - 1
NBUF = 6

_T_DN = (((0,), (0,)), ((), ()))


def _copy(a_hbm, bufs, sems, b, h, slot):
    return pltpu.make_async_copy(
        a_hbm.at[b, h], bufs.at[slot], sems.at[slot]
    )


def _merge_kernel(a_hbm, tokens_ref, merged_ref, idx_ref, bufs, accA, accB, m_ref, sems):
    b = pl.program_id(0)

    @pl.when(b == 0)
    def _prologue():
        for h in range(NBUF):
            _copy(a_hbm, bufs, sems, 0, h, h).start()

    def wait(h):
        _copy(a_hbm, bufs, sems, b, h, h % NBUF).wait()

    def reissue(h):
        nxt = h + NBUF
        if nxt < H:
            _copy(a_hbm, bufs, sems, b, nxt, h % NBUF).start()
        else:
            @pl.when(b < B - 1)
            def _n():
                _copy(a_hbm, bufs, sems, b + 1, nxt - H, h % NBUF).start()

    wait(0)
    accA[...] = bufs[0]
    reissue(0)
    wait(1)
    wait(2)
    accA[...] = accA[...] + (bufs[1] + bufs[2])
    reissue(1)
    reissue(2)
    for h in (3, 4, 5):
        wait(h)
        accA[...] = accA[...] + bufs[h % NBUF]
        reissue(h)
    wait(6)
    accB[...] = bufs[0]
    reissue(6)
    wait(7)
    wait(8)
    accB[...] = accB[...] + (bufs[1] + bufs[2])
    reissue(7)
    reissue(8)
    for h in (9, 10, 11):
        wait(h)
        accB[...] = accB[...] + bufs[h % NBUF]
        reissue(h)

    m_ref[...] = (accA[...] + accB[...]) * (jnp.float32(1) / jnp.float32(H))

    mass = m_ref[0:1, :]
    for q in range(1, N):
        mass = mass + m_ref[q:q + 1, :]

    pw = mass[:, 1:N]
    ones = jnp.ones((1, NP), jnp.float32)
    vcol = jax.lax.dot_general(
        pw, ones, _T_DN,
        precision=jax.lax.Precision.HIGHEST,
        preferred_element_type=jnp.float32,
    )
    vrow = jnp.broadcast_to(pw, (NP, NP))
    jj = jax.lax.broadcasted_iota(jnp.int32, (NP, NP), 1)
    ii = jax.lax.broadcasted_iota(jnp.int32, (NP, NP), 0)
    beats = (vrow > vcol) | ((vrow == vcol) & (jj < ii))
    rank = jnp.sum(beats.astype(jnp.int32), axis=1, keepdims=True)

    rr = jax.lax.broadcasted_iota(jnp.int32, (NP, K), 1)
    sel_mask = rank == rr
    iidx = jax.lax.broadcasted_iota(jnp.int32, (NP, K), 0)
    idx_ref[0, :] = jnp.sum(jnp.where(sel_mask, iidx, 0), axis=0)[None, :]

    mask16 = sel_mask.astype(jnp.bfloat16)
    patches = tokens_ref[0, 1:N, :]
    hi = patches.astype(jnp.bfloat16)
    rest = (patches - hi.astype(jnp.float32)).astype(jnp.bfloat16)
    sel = jax.lax.dot_general(
        mask16, hi, _T_DN, preferred_element_type=jnp.float32,
    ) + jax.lax.dot_general(
        mask16, rest, _T_DN, preferred_element_type=jnp.float32,
    )
    merged_ref[0, 0] = tokens_ref[0, 0]
    merged_ref[0, 1:K + 1, :] = sel


@jax.jit
def kernel(tokens, attention_maps):
    merged, idx = pl.pallas_call(
        _merge_kernel,
        grid=(B,),
        in_specs=[
            pl.BlockSpec(memory_space=pl.ANY),
            pl.BlockSpec((1, N, D), lambda b: (b, 0, 0)),
        ],
        out_specs=[
            pl.BlockSpec((1, K + 1, D), lambda b: (b, 0, 0)),
            pl.BlockSpec((1, 1, K), lambda b: (b, 0, 0)),
        ],
        out_shape=[
            jax.ShapeDtypeStruct((B, K + 1, D), jnp.float32),
            jax.ShapeDtypeStruct((B, 1, K), jnp.int32),
        ],
        scratch_shapes=[
            pltpu.VMEM((NBUF, N, N), jnp.float32),
            pltpu.VMEM((N, N), jnp.float32),
            pltpu.VMEM((N, N), jnp.float32),
            pltpu.VMEM((N, N), jnp.float32),
            pltpu.SemaphoreType.DMA((NBUF,)),
        ],
        compiler_params=pltpu.CompilerParams(
            dimension_semantics=("arbitrary",),
        ),
    )(attention_maps, tokens)
    return merged, idx.reshape(B, K)


# speedup vs baseline: 1.0217x; 1.0217x over previous
import jax
import jax.numpy as jnp
from jax.experimental import pallas as pl
from jax.experimental.pallas import tpu as pltpu

B, H, N, D = 8, 12, 577, 768
K = 288
NP = N - 1
HH = 6

_T_DN = (((0,), (0,)), ((), ()))


def _merge_kernel(w_ref, a_hbm, tokens_ref, merged_ref, idx_ref, bufs, m_ref, sems):
    b = pl.program_id(0)

    def chunk(bb, c, slot):
        return pltpu.make_async_copy(
            a_hbm.at[bb, pl.ds(HH + 3 * c, 3)], bufs.at[slot], sems.at[slot]
        )

    @pl.when(b == 0)
    def _prologue():
        chunk(0, 0, 0).start()
        chunk(0, 1, 1).start()

    w = w_ref[0]
    half1 = (((w[0] + (w[1] + w[2])) + w[3]) + w[4]) + w[5]

    chunk(b, 0, 0).wait()
    c0 = bufs[0]
    accB = c0[0] + (c0[1] + c0[2])
    chunk(b, 1, 1).wait()
    c1 = bufs[1]
    half2 = ((accB + c1[0]) + c1[1]) + c1[2]

    @pl.when(b < B - 1)
    def _reissue():
        chunk(b + 1, 0, 0).start()
        chunk(b + 1, 1, 1).start()

    m_ref[...] = (half1 + half2) * (jnp.float32(1) / jnp.float32(H))

    mass = m_ref[0:1, :]
    for q in range(1, N):
        mass = mass + m_ref[q:q + 1, :]

    pw = mass[:, 1:N]
    ones = jnp.ones((1, NP), jnp.float32)
    vcol = jax.lax.dot_general(
        pw, ones, _T_DN,
        precision=jax.lax.Precision.HIGHEST,
        preferred_element_type=jnp.float32,
    )
    vrow = jnp.broadcast_to(pw, (NP, NP))
    jj = jax.lax.broadcasted_iota(jnp.int32, (NP, NP), 1)
    ii = jax.lax.broadcasted_iota(jnp.int32, (NP, NP), 0)
    beats = (vrow > vcol) | ((vrow == vcol) & (jj < ii))
    rank = jnp.sum(beats.astype(jnp.int32), axis=1, keepdims=True)

    rr = jax.lax.broadcasted_iota(jnp.int32, (NP, K), 1)
    sel_mask = rank == rr
    iidx = jax.lax.broadcasted_iota(jnp.int32, (NP, K), 0)
    idx_ref[0, :] = jnp.sum(jnp.where(sel_mask, iidx, 0), axis=0)[None, :]

    mask16 = sel_mask.astype(jnp.bfloat16)
    patches = tokens_ref[0, 1:N, :]
    hi = patches.astype(jnp.bfloat16)
    rest = (patches - hi.astype(jnp.float32)).astype(jnp.bfloat16)
    sel = jax.lax.dot_general(
        mask16, hi, _T_DN, preferred_element_type=jnp.float32,
    ) + jax.lax.dot_general(
        mask16, rest, _T_DN, preferred_element_type=jnp.float32,
    )
    merged_ref[0, 0] = tokens_ref[0, 0]
    merged_ref[0, 1:K + 1, :] = sel


@jax.jit
def kernel(tokens, attention_maps):
    merged, idx = pl.pallas_call(
        _merge_kernel,
        grid=(B,),
        in_specs=[
            pl.BlockSpec((1, HH, N, N), lambda b: (b, 0, 0, 0)),
            pl.BlockSpec(memory_space=pl.ANY),
            pl.BlockSpec((1, N, D), lambda b: (b, 0, 0)),
        ],
        out_specs=[
            pl.BlockSpec((1, K + 1, D), lambda b: (b, 0, 0)),
            pl.BlockSpec((1, 1, K), lambda b: (b, 0, 0)),
        ],
        out_shape=[
            jax.ShapeDtypeStruct((B, K + 1, D), jnp.float32),
            jax.ShapeDtypeStruct((B, 1, K), jnp.int32),
        ],
        scratch_shapes=[
            pltpu.VMEM((2, 3, N, N), jnp.float32),
            pltpu.VMEM((N, N), jnp.float32),
            pltpu.SemaphoreType.DMA((2,)),
        ],
        compiler_params=pltpu.CompilerParams(
            dimension_semantics=("arbitrary",),
        ),
    )(attention_maps, attention_maps, tokens)
    return merged, idx.reshape(B, K)
